# Initial kernel scaffold; baseline (speedup 1.0000x reference)
import jax
import jax.numpy as jnp
from jax.experimental import pallas as pl


def kernel(coord, elems, ind_1, elem_embed, W1, b1, W2, b2, W3, b3):
    raise NotImplementedError("write your pallas kernel here")


# speedup vs baseline: 4.6181x; 4.6181x over previous
import functools

import jax
import jax.numpy as jnp
from jax.experimental import pallas as pl

N_ATOMS = 16384
N_STRUCT = 16
N_ELEM = 95
EMB = 64
HID = 256

BLOCK = 2048


def _fused_body(coord_ref, elems_ref, ind_ref, emb_ref, w1_ref, b1_ref,
                w2_ref, b2_ref, w3_ref, b3_ref, out_ref):
    b = coord_ref.shape[0]
    elems = elems_ref[0, 0, :]
    ind = ind_ref[0, 0, :]

    onehot = (jax.lax.broadcasted_iota(jnp.int32, (b, N_ELEM), 1)
              == elems[:, None]).astype(jnp.float32)
    feat = jnp.dot(onehot, emb_ref[...], preferred_element_type=jnp.float32)

    h = jnp.concatenate([feat, coord_ref[...]], axis=1)
    h = jnp.tanh(jnp.dot(h, w1_ref[...], preferred_element_type=jnp.float32)
                 + b1_ref[0, :])
    h = jnp.tanh(jnp.dot(h, w2_ref[...], preferred_element_type=jnp.float32)
                 + b2_ref[0, :])
    per_atom = (jnp.dot(h, w3_ref[...], preferred_element_type=jnp.float32)
                + b3_ref[0, 0])

    seg = (jax.lax.broadcasted_iota(jnp.int32, (b, N_STRUCT), 1)
           == ind[:, None]).astype(jnp.float32)
    part = jnp.dot(per_atom.reshape(1, b), seg,
                   preferred_element_type=jnp.float32)

    @pl.when(pl.program_id(0) == 0)
    def _init():
        out_ref[...] = part

    @pl.when(pl.program_id(0) != 0)
    def _acc():
        out_ref[...] += part


@jax.jit
def kernel(coord, elems, ind_1, elem_embed, W1, b1, W2, b2, W3, b3):
    n = coord.shape[0]
    grid = n // BLOCK
    elems3 = elems.astype(jnp.int32).reshape(grid, 1, BLOCK)
    ind3 = ind_1.astype(jnp.int32).reshape(grid, 1, BLOCK)

    out = pl.pallas_call(
        _fused_body,
        grid=(grid,),
        in_specs=[
            pl.BlockSpec((BLOCK, 3), lambda i: (i, 0)),
            pl.BlockSpec((1, 1, BLOCK), lambda i: (i, 0, 0)),
            pl.BlockSpec((1, 1, BLOCK), lambda i: (i, 0, 0)),
            pl.BlockSpec((N_ELEM, EMB), lambda i: (0, 0)),
            pl.BlockSpec((EMB + 3, HID), lambda i: (0, 0)),
            pl.BlockSpec((1, HID), lambda i: (0, 0)),
            pl.BlockSpec((HID, HID), lambda i: (0, 0)),
            pl.BlockSpec((1, HID), lambda i: (0, 0)),
            pl.BlockSpec((HID, 1), lambda i: (0, 0)),
            pl.BlockSpec((1, 1), lambda i: (0, 0)),
        ],
        out_specs=pl.BlockSpec((1, N_STRUCT), lambda i: (0, 0)),
        out_shape=jax.ShapeDtypeStruct((1, N_STRUCT), jnp.float32),
    )(coord, elems3, ind3, elem_embed, W1, b1.reshape(1, HID), W2,
      b2.reshape(1, HID), W3, b3.reshape(1, 1))
    return out[0]
